# Initial kernel scaffold; baseline (speedup 1.0000x reference)
import jax
import jax.numpy as jnp
from jax.experimental import pallas as pl


def kernel(x, lap_indices, lap_values, params):
    raise NotImplementedError("write your pallas kernel here")


# speedup vs baseline: 2.1788x; 2.1788x over previous
import functools

import jax
import jax.numpy as jnp
import numpy as np
from jax import lax
from jax.experimental import pallas as pl
from jax.experimental.pallas import tpu as pltpu
from jax.experimental.pallas import tpu_sc as plsc

_V = 10000
_E = 160000
_B = 2
_NC = 2
_NS = 16
_RPT = 624
_EPT = _E // _NS
_CHUNK = 80
_NCHUNK = _EPT // _CHUNK

_GDN = jax.lax.GatherDimensionNumbers(
    offset_dims=(), collapsed_slice_dims=(0,), start_index_map=(0,))


@functools.lru_cache(maxsize=None)
def _make_spmm(C):
    FS = C // 16
    mesh = plsc.VectorSubcoreMesh(core_axis_name="c", subcore_axis_name="s",
                                  num_cores=_NC, num_subcores=_NS)

    @functools.partial(
        pl.kernel,
        out_type=jax.ShapeDtypeStruct((_B * _V, C), jnp.float32),
        mesh=mesh,
        compiler_params=pltpu.CompilerParams(use_tc_tiling_on_sc=False),
        scratch_types=[
            pltpu.VMEM((_CHUNK,), jnp.int32),
            pltpu.VMEM((_CHUNK,), jnp.int32),
            pltpu.VMEM((_CHUNK, C), jnp.float32),
            pltpu.VMEM((_CHUNK,), jnp.float32),
            pltpu.VMEM_SHARED((_V, C), jnp.float32),
        ],
    )
    def spmm(xt, rows, cols, vals, out, cidx, ridx, gbuf, vstage, ysh):
        b = lax.axis_index("c")
        s = lax.axis_index("s")
        zero16 = jnp.zeros((16,), jnp.float32)

        def _zrow(j, _):
            for f in range(FS):
                gbuf[j, pl.ds(16 * f, 16)] = zero16
            return 0
        lax.fori_loop(0, _CHUNK, _zrow, 0)
        rbase = s * _RPT
        off = 0
        while off < _RPT:
            n = min(_CHUNK, _RPT - off)
            pltpu.sync_copy(gbuf.at[pl.ds(0, n)], ysh.at[pl.ds(rbase + off, n)])
            off += n

        @pl.when(s == _NS - 1)
        def _():
            pltpu.sync_copy(gbuf.at[pl.ds(0, _V - _NS * _RPT)],
                            ysh.at[pl.ds(_NS * _RPT, _V - _NS * _RPT)])
        plsc.subcore_barrier()

        ebase = s * _EPT
        boff = jnp.full((16,), b * _V, jnp.int32)

        def chunk(g, _):
            e0 = ebase + g * _CHUNK
            pltpu.sync_copy(cols.at[pl.ds(e0, _CHUNK)], cidx)
            pltpu.sync_copy(rows.at[pl.ds(e0, _CHUNK)], ridx)
            pltpu.sync_copy(vals.at[pl.ds(e0, _CHUNK)], vstage)
            for q in range(_CHUNK // 16):
                sl = pl.ds(16 * q, 16)
                cidx[sl] = cidx[sl] + boff
            pltpu.sync_copy(xt.at[cidx], gbuf)

            def egroup(gg, _):
                vv16 = vstage[pl.ds(16 * gg, 16)]
                for lane in range(16):
                    lidx = jnp.full((16, 1), lane, jnp.int32)
                    vv = lax.gather(
                        vv16, lidx, _GDN, slice_sizes=(1,),
                        mode=lax.GatherScatterMode.PROMISE_IN_BOUNDS)
                    j = 16 * gg + lane
                    for f in range(FS):
                        sl = pl.ds(16 * f, 16)
                        gbuf[j, sl] = gbuf[j, sl] * vv
                return 0
            lax.fori_loop(0, _CHUNK // 16, egroup, 0)
            pltpu.sync_copy(gbuf, ysh.at[ridx], add=True)
            return 0
        lax.fori_loop(0, _NCHUNK, chunk, 0)

        plsc.subcore_barrier()
        pltpu.sync_copy(ysh.at[pl.ds(rbase, _RPT)],
                        out.at[pl.ds(b * _V + rbase, _RPT)])

        @pl.when(s == _NS - 1)
        def _():
            tail = _V - _NS * _RPT
            pltpu.sync_copy(ysh.at[pl.ds(_NS * _RPT, tail)],
                            out.at[pl.ds(b * _V + _NS * _RPT, tail)])

    return spmm


_VB = 2000
_DN = (((1,), (1,)), ((), ()))


def _tc_stats(z):
    B_, V_, C_ = z.shape

    def body(z_ref, o_ref):
        @pl.when(pl.program_id(0) == 0)
        def _():
            o_ref[...] = jnp.zeros_like(o_ref)
        zb = z_ref[...]
        o_ref[...] += jnp.stack(
            [jnp.sum(zb, axis=(0, 1)), jnp.sum(zb * zb, axis=(0, 1))])

    return pl.pallas_call(
        body,
        grid=(V_ // _VB,),
        in_specs=[pl.BlockSpec((B_, _VB, C_), lambda i: (0, i, 0))],
        out_specs=pl.BlockSpec((2, C_), lambda i: (0, 0)),
        out_shape=jax.ShapeDtypeStruct((2, C_), jnp.float32),
    )(z)


def _tc_bn_relu(z, st, gamma, beta):
    B_, V_, C_ = z.shape
    n = float(B_ * V_)

    def body(z_ref, st_ref, g_ref, b_ref, o_ref):
        mu = st_ref[0, :] * (1.0 / n)
        var = st_ref[1, :] * (1.0 / n) - mu * mu
        scale = g_ref[...] * lax.rsqrt(var + 1e-5)
        shift = b_ref[...] - mu * scale
        o_ref[...] = jnp.maximum(
            z_ref[...] * scale[None, None, :] + shift[None, None, :], 0.0)

    return pl.pallas_call(
        body,
        grid=(V_ // _VB,),
        in_specs=[
            pl.BlockSpec((B_, _VB, C_), lambda i: (0, i, 0)),
            pl.BlockSpec((2, C_), lambda i: (0, 0)),
            pl.BlockSpec((C_,), lambda i: (0,)),
            pl.BlockSpec((C_,), lambda i: (0,)),
        ],
        out_specs=pl.BlockSpec((B_, _VB, C_), lambda i: (0, i, 0)),
        out_shape=jax.ShapeDtypeStruct((B_, V_, C_), jnp.float32),
    )(z, st, gamma, beta)


def _tc_combine(h, t1, u, W, bias, ws=None, zres=None):
    B_, V_, ci = h.shape
    co = W.shape[1]

    def body(*refs):
        if ws is not None:
            h_ref, t1_ref, u_ref, w_ref, b_ref, e_ref, o_ref = refs
        elif zres is not None:
            h_ref, t1_ref, u_ref, w_ref, b_ref, e_ref, o_ref = refs
        else:
            h_ref, t1_ref, u_ref, w_ref, b_ref, o_ref = refs
        for bb in range(B_):
            hb = h_ref[bb]
            t2b = 2.0 * u_ref[bb] - hb
            acc = lax.dot_general(hb, w_ref[0], _DN,
                                  preferred_element_type=jnp.float32)
            acc += lax.dot_general(t1_ref[bb], w_ref[1], _DN,
                                   preferred_element_type=jnp.float32)
            acc += lax.dot_general(t2b, w_ref[2], _DN,
                                   preferred_element_type=jnp.float32)
            if ws is not None:
                acc += lax.dot_general(hb, e_ref[...], _DN,
                                       preferred_element_type=jnp.float32)
            elif zres is not None:
                acc += e_ref[bb]
            o_ref[bb] = acc + b_ref[...][None, :]

    in_specs = [
        pl.BlockSpec((B_, _VB, ci), lambda i: (0, i, 0)),
        pl.BlockSpec((B_, _VB, ci), lambda i: (0, i, 0)),
        pl.BlockSpec((B_, _VB, ci), lambda i: (0, i, 0)),
        pl.BlockSpec((3, co, ci), lambda i: (0, 0, 0)),
        pl.BlockSpec((co,), lambda i: (0,)),
    ]
    args = [h, t1, u, W, bias]
    if ws is not None:
        in_specs.append(pl.BlockSpec((co, ci), lambda i: (0, 0)))
        args.append(ws)
    elif zres is not None:
        in_specs.append(pl.BlockSpec((B_, _VB, co), lambda i: (0, i, 0)))
        args.append(zres)

    return pl.pallas_call(
        body,
        grid=(V_ // _VB,),
        in_specs=in_specs,
        out_specs=pl.BlockSpec((B_, _VB, co), lambda i: (0, i, 0)),
        out_shape=jax.ShapeDtypeStruct((B_, V_, co), jnp.float32),
    )(*args)


def _tc_head(z, fc_w, fc_b):
    B_, V_, C_ = z.shape
    ng = V_ // _VB
    nc = fc_w.shape[0]

    def body(z_ref, w_ref, b_ref, o_ref, acc):
        i = pl.program_id(0)

        @pl.when(i == 0)
        def _():
            acc[...] = jnp.full((B_, C_), -jnp.inf, jnp.float32)
        acc[...] = jnp.maximum(acc[...], jnp.max(z_ref[...], axis=1))

        @pl.when(i == ng - 1)
        def _():
            logits = lax.dot_general(
                acc[...], w_ref[...], _DN,
                preferred_element_type=jnp.float32) + b_ref[...][None, :]
            m = jnp.max(logits, axis=1, keepdims=True)
            lse = jnp.log(jnp.sum(jnp.exp(logits - m), axis=1,
                                  keepdims=True)) + m
            o_ref[...] = logits - lse

    return pl.pallas_call(
        body,
        grid=(ng,),
        in_specs=[
            pl.BlockSpec((B_, _VB, C_), lambda i: (0, i, 0)),
            pl.BlockSpec((nc, C_), lambda i: (0, 0)),
            pl.BlockSpec((nc,), lambda i: (0,)),
        ],
        out_specs=pl.BlockSpec((B_, nc), lambda i: (0, 0)),
        out_shape=jax.ShapeDtypeStruct((B_, nc), jnp.float32),
        scratch_shapes=[pltpu.VMEM((B_, C_), jnp.float32)],
    )(z, fc_w, fc_b)


def kernel(x, lap_indices, lap_values, params):
    rows = lap_indices[0].astype(jnp.int32)
    cols = lap_indices[1].astype(jnp.int32)
    vals = lap_values

    xt = jnp.transpose(x, (0, 2, 1))

    def flat(a):
        return a.reshape(_B * _V, a.shape[-1])

    def un(a):
        return a.reshape(_B, _V, a.shape[-1])

    def cheb_pair(hf):
        t1 = _make_spmm(hf.shape[-1])(hf, rows, cols, vals)
        u = _make_spmm(hf.shape[-1])(t1, rows, cols, vals)
        return t1, u

    xf = flat(xt)
    t1, u = cheb_pair(xf)
    out = _tc_combine(xt, un(t1), un(u), params['conv0_W'], params['conv0_b'])

    for blk in ('block1', 'block2', 'block3'):
        for j in (0, 1):
            lp = params[blk]['l%d' % j]
            z = out
            st = _tc_stats(z)
            h = _tc_bn_relu(z, st, lp['gamma'], lp['beta'])
            t1, u = cheb_pair(flat(h))
            if 'Ws' in lp:
                out = _tc_combine(h, un(t1), un(u), lp['W'], lp['b'],
                                  ws=lp['Ws'])
            else:
                out = _tc_combine(h, un(t1), un(u), lp['W'], lp['b'], zres=z)

    return _tc_head(out, params['fc_W'], params['fc_b'])


# speedup vs baseline: 5.0083x; 2.2986x over previous
import functools

import jax
import jax.numpy as jnp
import numpy as np
from jax import lax
from jax.experimental import pallas as pl
from jax.experimental.pallas import tpu as pltpu
from jax.experimental.pallas import tpu_sc as plsc

_V = 10000
_E = 160000
_B = 2
_NC = 2
_NS = 16
_RPT = 624
_EPT = _E // _NS
_CHUNK = 80
_NCHUNK = _EPT // _CHUNK

_GDN = jax.lax.GatherDimensionNumbers(
    offset_dims=(), collapsed_slice_dims=(0,), start_index_map=(0,))


@functools.lru_cache(maxsize=None)
def _make_spmm(C):
    FS = C // 16
    mesh = plsc.VectorSubcoreMesh(core_axis_name="c", subcore_axis_name="s",
                                  num_cores=_NC, num_subcores=_NS)

    @functools.partial(
        pl.kernel,
        out_type=jax.ShapeDtypeStruct((_B * _V, C), jnp.float32),
        mesh=mesh,
        compiler_params=pltpu.CompilerParams(use_tc_tiling_on_sc=False),
        scratch_types=[
            pltpu.VMEM((_NCHUNK, _CHUNK), jnp.int32),
            pltpu.VMEM((_NCHUNK, _CHUNK), jnp.int32),
            pltpu.VMEM((_NCHUNK, _CHUNK), jnp.float32),
            pltpu.VMEM((2, _CHUNK, C), jnp.float32),
            pltpu.VMEM_SHARED((_V, C), jnp.float32),
            pltpu.SemaphoreType.DMA,
            pltpu.SemaphoreType.DMA,
            pltpu.SemaphoreType.DMA,
            pltpu.SemaphoreType.DMA,
        ],
    )
    def spmm(xt, rows3, cols3, vals3, out, cbuf, rbuf, vbuf, gbuf, ysh,
             sg0, sg1, ss0, ss1):
        b = lax.axis_index("c")
        s = lax.axis_index("s")
        zero16 = jnp.zeros((16,), jnp.float32)
        sems_g = (sg0, sg1)
        sems_s = (ss0, ss1)

        pltpu.sync_copy(cols3.at[s], cbuf)
        pltpu.sync_copy(rows3.at[s], rbuf)
        pltpu.sync_copy(vals3.at[s], vbuf)

        boff = jnp.full((16,), b * _V, jnp.int32)

        def addoff(i, _):
            for q in range(_CHUNK // 16):
                sl = pl.ds(16 * q, 16)
                cbuf[i, sl] = cbuf[i, sl] + boff
            return 0
        lax.fori_loop(0, _NCHUNK, addoff, 0)

        def _zrow(j, _):
            for f in range(FS):
                gbuf[0, j, pl.ds(16 * f, 16)] = zero16
            return 0
        lax.fori_loop(0, _CHUNK, _zrow, 0)
        rbase = s * _RPT
        off = 0
        while off < _RPT:
            n = min(_CHUNK, _RPT - off)
            pltpu.sync_copy(gbuf.at[0, pl.ds(0, n)],
                            ysh.at[pl.ds(rbase + off, n)])
            off += n

        @pl.when(s == _NS - 1)
        def _():
            pltpu.sync_copy(gbuf.at[0, pl.ds(0, _V - _NS * _RPT)],
                            ysh.at[pl.ds(_NS * _RPT, _V - _NS * _RPT)])
        plsc.subcore_barrier()

        def start_gather(g, k):
            pltpu.async_copy(xt.at[cbuf.at[g]], gbuf.at[k], sems_g[k])

        def wait_gather(k):
            pltpu.make_async_copy(xt.at[cbuf.at[0]], gbuf.at[k],
                                  sems_g[k]).wait()

        def start_scatter(g, k):
            pltpu.async_copy(gbuf.at[k], ysh.at[rbuf.at[g]], sems_s[k],
                             add=True)

        def wait_scatter(k):
            pltpu.make_async_copy(gbuf.at[k], ysh.at[rbuf.at[0]],
                                  sems_s[k]).wait()

        def scale(g, k):
            def egroup(gg, _):
                vv16 = vbuf[g, pl.ds(16 * gg, 16)]
                for lane in range(16):
                    lidx = jnp.full((16, 1), lane, jnp.int32)
                    vv = lax.gather(
                        vv16, lidx, _GDN, slice_sizes=(1,),
                        mode=lax.GatherScatterMode.PROMISE_IN_BOUNDS)
                    j = 16 * gg + lane
                    for f in range(FS):
                        sl = pl.ds(16 * f, 16)
                        gbuf[k, j, sl] = gbuf[k, j, sl] * vv
                return 0
            lax.fori_loop(0, _CHUNK // 16, egroup, 0)

        start_gather(0, 0)

        def pair(i, _):
            g0 = 2 * i
            wait_gather(0)

            @pl.when(i > 0)
            def _():
                wait_scatter(1)
            start_gather(g0 + 1, 1)
            scale(g0, 0)
            start_scatter(g0, 0)
            wait_gather(1)
            wait_scatter(0)
            start_gather(g0 + 2, 0)
            scale(g0 + 1, 1)
            start_scatter(g0 + 1, 1)
            return 0
        lax.fori_loop(0, _NCHUNK // 2, pair, 0)

        wait_gather(0)
        wait_scatter(1)
        scale(_NCHUNK - 1, 0)
        start_scatter(_NCHUNK - 1, 0)
        wait_scatter(0)

        plsc.subcore_barrier()
        pltpu.sync_copy(ysh.at[pl.ds(rbase, _RPT)],
                        out.at[pl.ds(b * _V + rbase, _RPT)])

        @pl.when(s == _NS - 1)
        def _():
            tail = _V - _NS * _RPT
            pltpu.sync_copy(ysh.at[pl.ds(_NS * _RPT, tail)],
                            out.at[pl.ds(b * _V + _NS * _RPT, tail)])

    return spmm


_VB = 2000
_DN = (((1,), (1,)), ((), ()))


def _tc_stats(z):
    B_, V_, C_ = z.shape

    def body(z_ref, o_ref):
        @pl.when(pl.program_id(0) == 0)
        def _():
            o_ref[...] = jnp.zeros_like(o_ref)
        zb = z_ref[...]
        o_ref[...] += jnp.stack(
            [jnp.sum(zb, axis=(0, 1)), jnp.sum(zb * zb, axis=(0, 1))])

    return pl.pallas_call(
        body,
        grid=(V_ // _VB,),
        in_specs=[pl.BlockSpec((B_, _VB, C_), lambda i: (0, i, 0))],
        out_specs=pl.BlockSpec((2, C_), lambda i: (0, 0)),
        out_shape=jax.ShapeDtypeStruct((2, C_), jnp.float32),
    )(z)


def _tc_bn_relu(z, st, gamma, beta):
    B_, V_, C_ = z.shape
    n = float(B_ * V_)

    def body(z_ref, st_ref, g_ref, b_ref, o_ref):
        mu = st_ref[0, :] * (1.0 / n)
        var = st_ref[1, :] * (1.0 / n) - mu * mu
        scale = g_ref[...] * lax.rsqrt(var + 1e-5)
        shift = b_ref[...] - mu * scale
        o_ref[...] = jnp.maximum(
            z_ref[...] * scale[None, None, :] + shift[None, None, :], 0.0)

    return pl.pallas_call(
        body,
        grid=(V_ // _VB,),
        in_specs=[
            pl.BlockSpec((B_, _VB, C_), lambda i: (0, i, 0)),
            pl.BlockSpec((2, C_), lambda i: (0, 0)),
            pl.BlockSpec((C_,), lambda i: (0,)),
            pl.BlockSpec((C_,), lambda i: (0,)),
        ],
        out_specs=pl.BlockSpec((B_, _VB, C_), lambda i: (0, i, 0)),
        out_shape=jax.ShapeDtypeStruct((B_, V_, C_), jnp.float32),
    )(z, st, gamma, beta)


def _tc_combine(h, t1, u, W, bias, ws=None, zres=None):
    B_, V_, ci = h.shape
    co = W.shape[1]

    def body(*refs):
        if ws is not None:
            h_ref, t1_ref, u_ref, w_ref, b_ref, e_ref, o_ref = refs
        elif zres is not None:
            h_ref, t1_ref, u_ref, w_ref, b_ref, e_ref, o_ref = refs
        else:
            h_ref, t1_ref, u_ref, w_ref, b_ref, o_ref = refs
        for bb in range(B_):
            hb = h_ref[bb]
            t2b = 2.0 * u_ref[bb] - hb
            acc = lax.dot_general(hb, w_ref[0], _DN,
                                  preferred_element_type=jnp.float32)
            acc += lax.dot_general(t1_ref[bb], w_ref[1], _DN,
                                   preferred_element_type=jnp.float32)
            acc += lax.dot_general(t2b, w_ref[2], _DN,
                                   preferred_element_type=jnp.float32)
            if ws is not None:
                acc += lax.dot_general(hb, e_ref[...], _DN,
                                       preferred_element_type=jnp.float32)
            elif zres is not None:
                acc += e_ref[bb]
            o_ref[bb] = acc + b_ref[...][None, :]

    in_specs = [
        pl.BlockSpec((B_, _VB, ci), lambda i: (0, i, 0)),
        pl.BlockSpec((B_, _VB, ci), lambda i: (0, i, 0)),
        pl.BlockSpec((B_, _VB, ci), lambda i: (0, i, 0)),
        pl.BlockSpec((3, co, ci), lambda i: (0, 0, 0)),
        pl.BlockSpec((co,), lambda i: (0,)),
    ]
    args = [h, t1, u, W, bias]
    if ws is not None:
        in_specs.append(pl.BlockSpec((co, ci), lambda i: (0, 0)))
        args.append(ws)
    elif zres is not None:
        in_specs.append(pl.BlockSpec((B_, _VB, co), lambda i: (0, i, 0)))
        args.append(zres)

    return pl.pallas_call(
        body,
        grid=(V_ // _VB,),
        in_specs=in_specs,
        out_specs=pl.BlockSpec((B_, _VB, co), lambda i: (0, i, 0)),
        out_shape=jax.ShapeDtypeStruct((B_, V_, co), jnp.float32),
    )(*args)


def _tc_head(z, fc_w, fc_b):
    B_, V_, C_ = z.shape
    ng = V_ // _VB
    nc = fc_w.shape[0]

    def body(z_ref, w_ref, b_ref, o_ref, acc):
        i = pl.program_id(0)

        @pl.when(i == 0)
        def _():
            acc[...] = jnp.full((B_, C_), -jnp.inf, jnp.float32)
        acc[...] = jnp.maximum(acc[...], jnp.max(z_ref[...], axis=1))

        @pl.when(i == ng - 1)
        def _():
            logits = lax.dot_general(
                acc[...], w_ref[...], _DN,
                preferred_element_type=jnp.float32) + b_ref[...][None, :]
            m = jnp.max(logits, axis=1, keepdims=True)
            lse = jnp.log(jnp.sum(jnp.exp(logits - m), axis=1,
                                  keepdims=True)) + m
            o_ref[...] = logits - lse

    return pl.pallas_call(
        body,
        grid=(ng,),
        in_specs=[
            pl.BlockSpec((B_, _VB, C_), lambda i: (0, i, 0)),
            pl.BlockSpec((nc, C_), lambda i: (0, 0)),
            pl.BlockSpec((nc,), lambda i: (0,)),
        ],
        out_specs=pl.BlockSpec((B_, nc), lambda i: (0, 0)),
        out_shape=jax.ShapeDtypeStruct((B_, nc), jnp.float32),
        scratch_shapes=[pltpu.VMEM((B_, C_), jnp.float32)],
    )(z, fc_w, fc_b)


def kernel(x, lap_indices, lap_values, params):
    rows = lap_indices[0].astype(jnp.int32).reshape(_NS, _NCHUNK, _CHUNK)
    cols = lap_indices[1].astype(jnp.int32).reshape(_NS, _NCHUNK, _CHUNK)
    vals = lap_values.reshape(_NS, _NCHUNK, _CHUNK)

    xt = jnp.transpose(x, (0, 2, 1))

    def flat(a):
        return a.reshape(_B * _V, a.shape[-1])

    def un(a):
        return a.reshape(_B, _V, a.shape[-1])

    def cheb_pair(hf):
        t1 = _make_spmm(hf.shape[-1])(hf, rows, cols, vals)
        u = _make_spmm(hf.shape[-1])(t1, rows, cols, vals)
        return t1, u

    xf = flat(xt)
    t1, u = cheb_pair(xf)
    out = _tc_combine(xt, un(t1), un(u), params['conv0_W'], params['conv0_b'])

    for blk in ('block1', 'block2', 'block3'):
        for j in (0, 1):
            lp = params[blk]['l%d' % j]
            z = out
            st = _tc_stats(z)
            h = _tc_bn_relu(z, st, lp['gamma'], lp['beta'])
            t1, u = cheb_pair(flat(h))
            if 'Ws' in lp:
                out = _tc_combine(h, un(t1), un(u), lp['W'], lp['b'],
                                  ws=lp['Ws'])
            else:
                out = _tc_combine(h, un(t1), un(u), lp['W'], lp['b'], zres=z)

    return _tc_head(out, params['fc_W'], params['fc_b'])


# speedup vs baseline: 6.3836x; 1.2746x over previous
import functools

import jax
import jax.numpy as jnp
import numpy as np
from jax import lax
from jax.experimental import pallas as pl
from jax.experimental.pallas import tpu as pltpu
from jax.experimental.pallas import tpu_sc as plsc

_V = 10000
_E = 160000
_B = 2
_NC = 2
_NS = 16
_RPT = 624
_EPT = _E // _NS
_CHUNK = 80
_NCHUNK = _EPT // _CHUNK

_GDN = jax.lax.GatherDimensionNumbers(
    offset_dims=(), collapsed_slice_dims=(0,), start_index_map=(0,))


def _chunk_edges(C):
    if C <= 16:
        return 2000
    if C <= 64:
        return 400
    return 80


@functools.lru_cache(maxsize=None)
def _make_spmm(C):
    FS = C // 16
    L = _chunk_edges(C)
    NCH = _EPT // L
    NGRP = L // 16
    RPT = (_V // _NS) // 8 * 8
    TAIL = _V - _NS * RPT
    mesh = plsc.VectorSubcoreMesh(core_axis_name="c", subcore_axis_name="s",
                                  num_cores=_NC, num_subcores=_NS)

    @functools.partial(
        pl.kernel,
        out_type=jax.ShapeDtypeStruct((_B * _V, C), jnp.float32),
        mesh=mesh,
        compiler_params=pltpu.CompilerParams(use_tc_tiling_on_sc=False),
        scratch_types=[
            pltpu.VMEM((NCH, L), jnp.int32),
            pltpu.VMEM((NCH, L), jnp.int32),
            pltpu.VMEM((NCH, L), jnp.float32),
            pltpu.VMEM((2, L, C), jnp.float32),
            pltpu.VMEM_SHARED((_V, C), jnp.float32),
            pltpu.SemaphoreType.DMA,
            pltpu.SemaphoreType.DMA,
            pltpu.SemaphoreType.DMA,
            pltpu.SemaphoreType.DMA,
        ],
    )
    def spmm(xt, rows3, cols3, vals3, out, cbuf, rbuf, vbuf, gbuf, ysh,
             sg0, sg1, ss0, ss1):
        b = lax.axis_index("c")
        s = lax.axis_index("s")
        zero16 = jnp.zeros((16,), jnp.float32)
        sems_g = (sg0, sg1)
        sems_s = (ss0, ss1)

        def fire_gather(g, k):
            pltpu.async_copy(xt.at[cbuf.at[g]], gbuf.at[k], sems_g[k])

        def wait_gather(k):
            pltpu.make_async_copy(xt.at[pl.ds(0, L)], gbuf.at[k],
                                  sems_g[k]).wait()

        def fire_scatter(g, k):
            pltpu.async_copy(gbuf.at[k], ysh.at[rbuf.at[g]], sems_s[k],
                             add=True)

        def wait_scatter(k):
            pltpu.make_async_copy(gbuf.at[k], ysh.at[rbuf.at[0]],
                                  sems_s[k]).wait()

        def scale(g, k, glo, ghi):
            def egroup(gg, _):
                vv16 = vbuf[g, pl.ds(16 * gg, 16)]
                for lane in range(16):
                    lidx = jnp.full((16, 1), lane, jnp.int32)
                    vv = lax.gather(
                        vv16, lidx, _GDN, slice_sizes=(1,),
                        mode=lax.GatherScatterMode.PROMISE_IN_BOUNDS)
                    j = 16 * gg + lane
                    for f in range(FS):
                        sl = pl.ds(16 * f, 16)
                        gbuf[k, j, sl] = gbuf[k, j, sl] * vv
                return 0
            lax.fori_loop(glo, ghi, egroup, 0)

        pltpu.sync_copy(cols3.at[s], cbuf)
        pltpu.sync_copy(rows3.at[s], rbuf)
        pltpu.sync_copy(vals3.at[s], vbuf)

        boff = jnp.full((16,), b * _V, jnp.int32)

        def addoff(i, _):
            for q in range(NGRP):
                sl = pl.ds(16 * q, 16)
                cbuf[i, sl] = cbuf[i, sl] + boff
            return 0
        lax.fori_loop(0, NCH, addoff, 0)

        def _zrow(j, _):
            for f in range(FS):
                gbuf[0, j, pl.ds(16 * f, 16)] = zero16
            return 0
        lax.fori_loop(0, L, _zrow, 0)
        rbase = s * RPT
        off = 0
        while off < RPT:
            n = min(L, RPT - off)
            pltpu.sync_copy(gbuf.at[0, pl.ds(0, n)],
                            ysh.at[pl.ds(rbase + off, n)])
            off += n

        if TAIL:
            @pl.when(s == _NS - 1)
            def _():
                pltpu.sync_copy(gbuf.at[0, pl.ds(0, TAIL)],
                                ysh.at[pl.ds(_NS * RPT, TAIL)])

        fire_gather(0, 0)
        plsc.subcore_barrier()

        def phase(g, k, i):
            wait_gather(k)
            scale(g, k, 0, NGRP // 2)
            if k == 0 and not isinstance(i, int):
                @pl.when(i > 0)
                def _():
                    wait_scatter(1 - k)
            else:
                wait_scatter(1 - k)
            fire_gather(g + 1, 1 - k)
            scale(g, k, NGRP // 2, NGRP)
            fire_scatter(g, k)

        def pair(i, _):
            phase(2 * i, 0, i)
            phase(2 * i + 1, 1, i)
            return 0
        lax.fori_loop(0, (NCH - 1) // 2, pair, 0)

        wait_gather(0)
        scale(NCH - 1, 0, 0, NGRP)
        fire_scatter(NCH - 1, 0)
        wait_scatter(1)
        wait_scatter(0)

        plsc.subcore_barrier()
        pltpu.sync_copy(ysh.at[pl.ds(rbase, RPT)],
                        out.at[pl.ds(b * _V + rbase, RPT)])

        if TAIL:
            @pl.when(s == _NS - 1)
            def _():
                pltpu.sync_copy(ysh.at[pl.ds(_NS * RPT, TAIL)],
                                out.at[pl.ds(b * _V + _NS * RPT, TAIL)])

    return spmm


_VB = 2000
_DN = (((1,), (1,)), ((), ()))


def _tc_stats(z):
    B_, V_, C_ = z.shape

    def body(z_ref, o_ref):
        @pl.when(pl.program_id(0) == 0)
        def _():
            o_ref[...] = jnp.zeros_like(o_ref)
        zb = z_ref[...]
        o_ref[...] += jnp.stack(
            [jnp.sum(zb, axis=(0, 1)), jnp.sum(zb * zb, axis=(0, 1))])

    return pl.pallas_call(
        body,
        grid=(V_ // _VB,),
        in_specs=[pl.BlockSpec((B_, _VB, C_), lambda i: (0, i, 0))],
        out_specs=pl.BlockSpec((2, C_), lambda i: (0, 0)),
        out_shape=jax.ShapeDtypeStruct((2, C_), jnp.float32),
    )(z)


def _tc_bn_relu(z, st, gamma, beta):
    B_, V_, C_ = z.shape
    n = float(B_ * V_)

    def body(z_ref, st_ref, g_ref, b_ref, o_ref):
        mu = st_ref[0, :] * (1.0 / n)
        var = st_ref[1, :] * (1.0 / n) - mu * mu
        scale = g_ref[...] * lax.rsqrt(var + 1e-5)
        shift = b_ref[...] - mu * scale
        o_ref[...] = jnp.maximum(
            z_ref[...] * scale[None, None, :] + shift[None, None, :], 0.0)

    return pl.pallas_call(
        body,
        grid=(V_ // _VB,),
        in_specs=[
            pl.BlockSpec((B_, _VB, C_), lambda i: (0, i, 0)),
            pl.BlockSpec((2, C_), lambda i: (0, 0)),
            pl.BlockSpec((C_,), lambda i: (0,)),
            pl.BlockSpec((C_,), lambda i: (0,)),
        ],
        out_specs=pl.BlockSpec((B_, _VB, C_), lambda i: (0, i, 0)),
        out_shape=jax.ShapeDtypeStruct((B_, V_, C_), jnp.float32),
    )(z, st, gamma, beta)


def _tc_combine(h, t1, u, W, bias, ws=None, zres=None):
    B_, V_, ci = h.shape
    co = W.shape[1]

    def body(*refs):
        if ws is not None:
            h_ref, t1_ref, u_ref, w_ref, b_ref, e_ref, o_ref = refs
        elif zres is not None:
            h_ref, t1_ref, u_ref, w_ref, b_ref, e_ref, o_ref = refs
        else:
            h_ref, t1_ref, u_ref, w_ref, b_ref, o_ref = refs
        for bb in range(B_):
            hb = h_ref[bb]
            t2b = 2.0 * u_ref[bb] - hb
            acc = lax.dot_general(hb, w_ref[0], _DN,
                                  preferred_element_type=jnp.float32)
            acc += lax.dot_general(t1_ref[bb], w_ref[1], _DN,
                                   preferred_element_type=jnp.float32)
            acc += lax.dot_general(t2b, w_ref[2], _DN,
                                   preferred_element_type=jnp.float32)
            if ws is not None:
                acc += lax.dot_general(hb, e_ref[...], _DN,
                                       preferred_element_type=jnp.float32)
            elif zres is not None:
                acc += e_ref[bb]
            o_ref[bb] = acc + b_ref[...][None, :]

    in_specs = [
        pl.BlockSpec((B_, _VB, ci), lambda i: (0, i, 0)),
        pl.BlockSpec((B_, _VB, ci), lambda i: (0, i, 0)),
        pl.BlockSpec((B_, _VB, ci), lambda i: (0, i, 0)),
        pl.BlockSpec((3, co, ci), lambda i: (0, 0, 0)),
        pl.BlockSpec((co,), lambda i: (0,)),
    ]
    args = [h, t1, u, W, bias]
    if ws is not None:
        in_specs.append(pl.BlockSpec((co, ci), lambda i: (0, 0)))
        args.append(ws)
    elif zres is not None:
        in_specs.append(pl.BlockSpec((B_, _VB, co), lambda i: (0, i, 0)))
        args.append(zres)

    return pl.pallas_call(
        body,
        grid=(V_ // _VB,),
        in_specs=in_specs,
        out_specs=pl.BlockSpec((B_, _VB, co), lambda i: (0, i, 0)),
        out_shape=jax.ShapeDtypeStruct((B_, V_, co), jnp.float32),
    )(*args)


def _tc_head(z, fc_w, fc_b):
    B_, V_, C_ = z.shape
    ng = V_ // _VB
    nc = fc_w.shape[0]

    def body(z_ref, w_ref, b_ref, o_ref, acc):
        i = pl.program_id(0)

        @pl.when(i == 0)
        def _():
            acc[...] = jnp.full((B_, C_), -jnp.inf, jnp.float32)
        acc[...] = jnp.maximum(acc[...], jnp.max(z_ref[...], axis=1))

        @pl.when(i == ng - 1)
        def _():
            logits = lax.dot_general(
                acc[...], w_ref[...], _DN,
                preferred_element_type=jnp.float32) + b_ref[...][None, :]
            m = jnp.max(logits, axis=1, keepdims=True)
            lse = jnp.log(jnp.sum(jnp.exp(logits - m), axis=1,
                                  keepdims=True)) + m
            o_ref[...] = logits - lse

    return pl.pallas_call(
        body,
        grid=(ng,),
        in_specs=[
            pl.BlockSpec((B_, _VB, C_), lambda i: (0, i, 0)),
            pl.BlockSpec((nc, C_), lambda i: (0, 0)),
            pl.BlockSpec((nc,), lambda i: (0,)),
        ],
        out_specs=pl.BlockSpec((B_, nc), lambda i: (0, 0)),
        out_shape=jax.ShapeDtypeStruct((B_, nc), jnp.float32),
        scratch_shapes=[pltpu.VMEM((B_, C_), jnp.float32)],
    )(z, fc_w, fc_b)


def kernel(x, lap_indices, lap_values, params):
    rows_f = lap_indices[0].astype(jnp.int32)
    cols_f = lap_indices[1].astype(jnp.int32)
    vals_f = lap_values
    prep_cache = {}

    def _prep_edges(C):
        L = _chunk_edges(C)
        if L not in prep_cache:
            NCH = _EPT // L
            prep_cache[L] = (rows_f.reshape(_NS, NCH, L),
                             cols_f.reshape(_NS, NCH, L),
                             vals_f.reshape(_NS, NCH, L))
        return prep_cache[L]

    xt = jnp.transpose(x, (0, 2, 1))

    def flat(a):
        return a.reshape(_B * _V, a.shape[-1])

    def un(a):
        return a.reshape(_B, _V, a.shape[-1])

    def spmm_call(z):
        C = z.shape[-1]
        r3, c3, v3 = _prep_edges(C)
        return _make_spmm(C)(z, r3, c3, v3)

    def cheb_pair(hf):
        t1 = spmm_call(hf)
        u = spmm_call(t1)
        return t1, u

    xf = flat(xt)
    t1, u = cheb_pair(xf)
    out = _tc_combine(xt, un(t1), un(u), params['conv0_W'], params['conv0_b'])

    for blk in ('block1', 'block2', 'block3'):
        for j in (0, 1):
            lp = params[blk]['l%d' % j]
            z = out
            st = _tc_stats(z)
            h = _tc_bn_relu(z, st, lp['gamma'], lp['beta'])
            t1, u = cheb_pair(flat(h))
            if 'Ws' in lp:
                out = _tc_combine(h, un(t1), un(u), lp['W'], lp['b'],
                                  ws=lp['Ws'])
            else:
                out = _tc_combine(h, un(t1), un(u), lp['W'], lp['b'], zres=z)

    return _tc_head(out, params['fc_W'], params['fc_b'])
